# baseline (speedup 1.0000x reference)
import functools

import jax
import jax.numpy as jnp
from jax import lax
from jax.experimental import pallas as pl
from jax.experimental.pallas import tpu as pltpu
from jax.experimental.pallas import tpu_sc as plsc

VOCAB = 1000000
MAXLEN = 200
EMBED = 64
BATCH = 4096
SEQ = 200

NUM_CORES = 2
NUM_SUBCORES = 16
NUM_WORKERS = NUM_CORES * NUM_SUBCORES
B_PER_W = BATCH // NUM_WORKERS
HALF = SEQ // 2
OUT_SLAB = EMBED * SEQ


@functools.partial(
    pl.kernel,
    out_type=jax.ShapeDtypeStruct((BATCH, OUT_SLAB), jnp.float32),
    mesh=plsc.VectorSubcoreMesh(core_axis_name="c", subcore_axis_name="s"),
    compiler_params=pltpu.CompilerParams(
        needs_layout_passes=False, use_tc_tiling_on_sc=False
    ),
    scratch_types=[
        pltpu.VMEM((2 * B_PER_W, HALF), jnp.int32),
        pltpu.VMEM((SEQ, EMBED), jnp.float32),
        pltpu.VMEM((2, HALF, EMBED), jnp.float32),
        pltpu.VMEM((OUT_SLAB,), jnp.float32),
        pltpu.SemaphoreType.DMA,
    ],
)
def _tpe_sc(x_hbm, tok_hbm, pos_hbm, out_hbm, idx_v, pos_v, row_v, out_v, sem):
    wid = lax.axis_index("s") * NUM_CORES + lax.axis_index("c")
    b0 = wid * B_PER_W

    pltpu.sync_copy(pos_hbm, pos_v)
    pltpu.sync_copy(x_hbm.at[pl.ds(b0 * 2, 2 * B_PER_W)], idx_v)

    lane = lax.iota(jnp.int32, 16)
    d_bases = [(d0 * 16 + lane) * SEQ for d0 in range(EMBED // 16)]

    def batch_body(j, carry):
        cp0 = pltpu.async_copy(tok_hbm.at[idx_v.at[2 * j]], row_v.at[0], sem)
        cp1 = pltpu.async_copy(tok_hbm.at[idx_v.at[2 * j + 1]], row_v.at[1], sem)
        cp0.wait()
        cp1.wait()

        def s_body(s100, carry2):
            for h in range(2):
                s = h * HALF + s100
                for d0 in range(EMBED // 16):
                    v = (row_v[h, s100, pl.ds(d0 * 16, 16)]
                         + pos_v[s, pl.ds(d0 * 16, 16)])
                    plsc.store_scatter(out_v, [d_bases[d0] + s], v)
            return carry2

        lax.fori_loop(0, HALF, s_body, 0, unroll=False)
        pltpu.sync_copy(out_v, out_hbm.at[b0 + j])
        return carry

    lax.fori_loop(0, B_PER_W, batch_body, 0, unroll=False)


def kernel(x, token_table, pos_table):
    x2 = x.astype(jnp.int32).reshape(BATCH * 2, HALF)
    out = _tpe_sc(x2, token_table, pos_table)
    return out.reshape(BATCH, EMBED, SEQ)


# speedup vs baseline: 1.5327x; 1.5327x over previous
import functools

import jax
import jax.numpy as jnp
from jax import lax
from jax.experimental import pallas as pl
from jax.experimental.pallas import tpu as pltpu
from jax.experimental.pallas import tpu_sc as plsc

VOCAB = 1000000
MAXLEN = 200
EMBED = 64
BATCH = 4096
SEQ = 200

NUM_CORES = 2
NUM_SUBCORES = 16
NUM_WORKERS = NUM_CORES * NUM_SUBCORES
B_PER_W = BATCH // NUM_WORKERS
HALF = SEQ // 2
OUT_SLAB = EMBED * SEQ
D_BLK = EMBED // 16


@functools.partial(
    pl.kernel,
    out_type=jax.ShapeDtypeStruct((BATCH, OUT_SLAB), jnp.float32),
    mesh=plsc.VectorSubcoreMesh(core_axis_name="c", subcore_axis_name="s"),
    compiler_params=pltpu.CompilerParams(
        needs_layout_passes=False, use_tc_tiling_on_sc=False
    ),
    scratch_types=[
        pltpu.VMEM((2 * B_PER_W, HALF), jnp.int32),
        pltpu.VMEM((SEQ, EMBED), jnp.float32),
        pltpu.VMEM((4, HALF, EMBED), jnp.float32),
        pltpu.VMEM((2, OUT_SLAB), jnp.float32),
        pltpu.SemaphoreType.DMA,
        pltpu.SemaphoreType.DMA,
        pltpu.SemaphoreType.DMA,
        pltpu.SemaphoreType.DMA,
    ],
)
def _tpe_sc(x_hbm, tok_hbm, pos_hbm, out_hbm, idx_v, pos_v, row_v, out_v,
            sg0, sg1, so0, so1):
    wid = lax.axis_index("s") * NUM_CORES + lax.axis_index("c")
    b0 = wid * B_PER_W
    sg = (sg0, sg1)
    so = (so0, so1)

    pltpu.sync_copy(pos_hbm, pos_v)
    pltpu.sync_copy(x_hbm.at[pl.ds(b0 * 2, 2 * B_PER_W)], idx_v)

    lane = lax.iota(jnp.int32, 16)
    d_bases = [
        [(d0 * 16 + lane) * SEQ + h * HALF for d0 in range(D_BLK)]
        for h in range(2)
    ]

    def issue_gathers(j, q):
        pltpu.async_copy(tok_hbm.at[idx_v.at[2 * j]], row_v.at[2 * q], sg[q])
        pltpu.async_copy(
            tok_hbm.at[idx_v.at[2 * j + 1]], row_v.at[2 * q + 1], sg[q]
        )

    def wait_gathers(q):
        for h in range(2):
            pltpu.make_async_copy(
                tok_hbm.at[pl.ds(0, HALF)], row_v.at[2 * q + h], sg[q]
            ).wait()

    def wait_out(p):
        pltpu.make_async_copy(out_v.at[p], out_hbm.at[b0], so[p]).wait()

    def transpose_into(p, q):
        ob = out_v.at[p]

        @plsc.parallel_loop(0, HALF, unroll=2)
        def s_body(s100):
            svec = jnp.full((16,), 0, jnp.int32) + s100
            for h in range(2):
                for d0 in range(D_BLK):
                    v = (row_v[2 * q + h, s100, pl.ds(d0 * 16, 16)]
                         + pos_v[h * HALF + s100, pl.ds(d0 * 16, 16)])
                    plsc.store_scatter(ob, [d_bases[h][d0] + svec], v)

    issue_gathers(0, 0)

    def pair_body(j2, carry):
        j = 2 * j2
        issue_gathers(j + 1, 1)
        wait_gathers(0)

        @pl.when(j2 > 0)
        def _():
            wait_out(0)

        transpose_into(0, 0)
        pltpu.async_copy(out_v.at[0], out_hbm.at[b0 + j], so[0])

        @pl.when(j2 < B_PER_W // 2 - 1)
        def _():
            issue_gathers(j + 2, 0)

        wait_gathers(1)

        @pl.when(j2 > 0)
        def _():
            wait_out(1)

        transpose_into(1, 1)
        pltpu.async_copy(out_v.at[1], out_hbm.at[b0 + j + 1], so[1])
        return carry

    lax.fori_loop(0, B_PER_W // 2, pair_body, 0, unroll=False)
    wait_out(0)
    wait_out(1)


def kernel(x, token_table, pos_table):
    x2 = x.astype(jnp.int32).reshape(BATCH * 2, HALF)
    out = _tpe_sc(x2, token_table, pos_table)
    return out.reshape(BATCH, EMBED, SEQ)


# speedup vs baseline: 2.2366x; 1.4592x over previous
import functools

import jax
import jax.numpy as jnp
from jax import lax
from jax.experimental import pallas as pl
from jax.experimental.pallas import tpu as pltpu
from jax.experimental.pallas import tpu_sc as plsc

VOCAB = 1000000
EMBED = 64
BATCH = 4096
SEQ = 200

NUM_CORES = 2
NUM_SUBCORES = 16
NUM_WORKERS = NUM_CORES * NUM_SUBCORES
SB = SEQ // 8
BB = BATCH // 128
CHUNK = 256
N_STEPS = SB * 1024 // CHUNK
OPAD = CHUNK + 1
D_BLK = EMBED // 16


@functools.partial(
    pl.kernel,
    out_type=jax.ShapeDtypeStruct((EMBED, SB, BB, 1024), jnp.float32),
    mesh=plsc.VectorSubcoreMesh(core_axis_name="c", subcore_axis_name="s"),
    compiler_params=pltpu.CompilerParams(
        needs_layout_passes=False, use_tc_tiling_on_sc=False
    ),
    scratch_types=[
        pltpu.VMEM((SB, 1024), jnp.int32),
        pltpu.VMEM((SEQ, EMBED), jnp.float32),
        pltpu.VMEM((2, CHUNK, 2, 32), jnp.float32),
        pltpu.VMEM((2, EMBED, OPAD), jnp.float32),
        pltpu.SemaphoreType.DMA,
        pltpu.SemaphoreType.DMA,
        pltpu.SemaphoreType.DMA,
        pltpu.SemaphoreType.DMA,
    ],
)
def _tpe_sc(x_hbm, tok_hbm, pos_hbm, out_hbm, idx_v, pos_v, row_v, out_v,
            sg0, sg1, so0, so1):
    wid = lax.axis_index("s") * NUM_CORES + lax.axis_index("c")
    sg = (sg0, sg1)
    so = (so0, so1)

    pltpu.sync_copy(pos_hbm, pos_v)
    pltpu.sync_copy(x_hbm.at[:, wid], idx_v)

    lane = lax.iota(jnp.int32, 16)
    d_rows = [d0 * 16 + lane for d0 in range(D_BLK)]

    def issue_gathers(step, q):
        sb = step // 4
        qq = step % 4
        for h in range(2):
            pltpu.async_copy(
                tok_hbm.at[idx_v.at[sb, pl.ds(qq * CHUNK + h * 128, 128)]],
                row_v.at[q, pl.ds(h * 128, 128)], sg[q])

    def wait_gathers(q):
        for h in range(2):
            pltpu.make_async_copy(
                tok_hbm.at[pl.ds(0, 128)],
                row_v.at[q, pl.ds(h * 128, 128)], sg[q]
            ).wait()

    def issue_out(step, p):
        sb = step // 4
        qq = step % 4
        pltpu.async_copy(
            out_v.at[p].at[:, pl.ds(0, CHUNK)],
            out_hbm.at[:, sb, wid].at[:, pl.ds(qq * CHUNK, CHUNK)], so[p])

    def wait_out(p):
        pltpu.make_async_copy(
            out_v.at[p].at[:, pl.ds(0, CHUNK)],
            out_hbm.at[:, 0, wid].at[:, pl.ds(0, CHUNK)], so[p]
        ).wait()

    def compute_into(step, p, q):
        ob = out_v.at[p]
        sb = step // 4
        qq = step % 4
        for h in range(2):
            s = 8 * sb + 2 * qq + h
            pvec = [pos_v[s, pl.ds(d0 * 16, 16)] for d0 in range(D_BLK)]

            @plsc.parallel_loop(0, 128, unroll=4)
            def bl_body(bl):
                r = h * 128 + bl
                cvec = jnp.full((16,), 0, jnp.int32) + r
                for d0 in range(D_BLK):
                    v = (row_v[q, r, d0 // 2, pl.ds((d0 % 2) * 16, 16)]
                         + pvec[d0])
                    plsc.store_scatter(ob, [d_rows[d0], cvec], v)

    issue_gathers(0, 0)

    def pair_body(j2, carry):
        step = 2 * j2
        issue_gathers(step + 1, 1)
        wait_gathers(0)

        @pl.when(j2 > 0)
        def _():
            wait_out(0)

        compute_into(step, 0, 0)
        issue_out(step, 0)

        @pl.when(j2 < N_STEPS // 2 - 1)
        def _():
            issue_gathers(step + 2, 0)

        wait_gathers(1)

        @pl.when(j2 > 0)
        def _():
            wait_out(1)

        compute_into(step + 1, 1, 1)
        issue_out(step + 1, 1)
        return carry

    lax.fori_loop(0, N_STEPS // 2, pair_body, 0, unroll=False)
    wait_out(0)
    wait_out(1)


def kernel(x, token_table, pos_table):
    x4 = (x.astype(jnp.int32).T.reshape(SB, 8, BB, 128)
          .transpose(0, 2, 1, 3).reshape(SB, BB, 1024))
    tok128 = jax.lax.optimization_barrier(token_table.reshape(VOCAB // 2, 128))
    tok3 = tok128.reshape(VOCAB, 2, 32)
    o = _tpe_sc(x4, tok3, pos_table)
    return (o.reshape(EMBED, SB, BB, 8, 128)
            .transpose(2, 4, 0, 1, 3).reshape(BATCH, EMBED, SEQ))
